# baseline (speedup 1.0000x reference)
import functools
import math

import jax
import jax.numpy as jnp
from jax import lax
from jax.experimental import pallas as pl
from jax.experimental.pallas import tpu as pltpu
from jax.experimental.pallas import tpu_sc as plsc

B = 16
L = 512
H = 256
M = 3
T_END = 100.0
TAUS = (1.0, 10.0, 100.0)
LNT = tuple(math.log(t) for t in TAUS)
C = 64
NCHUNK = L // C
GRID = NCHUNK + 1

_SC_INFO = plsc.get_sparse_core_info()
NW = _SC_INFO.num_cores * _SC_INFO.num_subcores
ROWS_W = 24
OUT_ROWS = NW * ROWS_W


def _delta_sc_body(sp_hbm, lens_hbm, out_hbm, sp_v, lens_v, buf_v):
    wid = lax.axis_index("s") * _SC_INFO.num_cores + lax.axis_index("c")
    lo = wid * ROWS_W
    pltpu.sync_copy(sp_hbm, sp_v)
    pltpu.sync_copy(lens_hbm, lens_v)
    lens = lens_v[...]
    lanes = lax.iota(jnp.int32, 16)
    t_last = jnp.zeros((16,), jnp.float32)
    for b in range(16):
        lb = lens[b]
        rowb = sp_v[lb - 1]
        oneh = jnp.clip(1 - jnp.abs(lanes - b), 0, 1).astype(jnp.float32)
        t_last = t_last + oneh * rowb
    fill_last = jnp.full((16,), T_END, jnp.float32) - t_last

    def row(idx, carry):
        j = lo + idx
        jc = jnp.minimum(j, L - 1)
        sp_j = sp_v[jc]
        sp_jm1 = sp_v[jnp.maximum(jc - 1, 0)]
        has_prev = jnp.minimum(j, 1)
        d = sp_j - jnp.full((16,), has_prev, jnp.int32).astype(jnp.float32) * sp_jm1
        a = lens - jnp.full((16,), j, jnp.int32)
        m_lt = jnp.clip(a, 0, 1).astype(jnp.float32)
        m_eq = jnp.clip(1 - jnp.abs(a), 0, 1).astype(jnp.float32)
        delta_j = m_lt * d + m_eq * fill_last - (1.0 - m_lt - m_eq)
        buf_v[idx] = delta_j
        return carry

    lax.fori_loop(0, ROWS_W, row, 0)
    pltpu.sync_copy(buf_v, out_hbm.at[pl.ds(lo, ROWS_W)])


_delta_sc = functools.partial(
    pl.kernel,
    out_type=jax.ShapeDtypeStruct((OUT_ROWS, B), jnp.float32),
    mesh=plsc.VectorSubcoreMesh(core_axis_name="c", subcore_axis_name="s"),
    scratch_types=[
        pltpu.VMEM((L, B), jnp.float32),
        pltpu.VMEM((B,), jnp.int32),
        pltpu.VMEM((ROWS_W, B), jnp.float32),
    ],
)(_delta_sc_body)


def _scan_body(delta_t_ref, whrs_ref, whq_ref, xb_ref,
               bef_ref, aft_ref, delta_ref, hhat_ref, dch_ref):
    i = pl.program_id(0)

    @pl.when(i == 0)
    def _init():
        cols = []
        for j in range(GRID):
            t = jnp.transpose(delta_t_ref[j * C:(j + 1) * C, :])
            dch_ref[j] = t
            cols.append(t)
        delta_ref[...] = jnp.concatenate(cols, axis=1)[:, :L + 1]
        hhat_ref[...] = jnp.zeros((M, B, H), jnp.float32)

    @pl.when(i < NCHUNK)
    def _chunk():
        dch = dch_ref[i]
        edt = [jnp.exp(dch * (-1.0 / TAUS[m])) for m in range(M)]
        h = [hhat_ref[m] for m in range(M)]
        wh_rs = whrs_ref[...]
        wh_q = whq_ref[...]
        wx_rs = xb_ref[0:1, :]
        b_rs = xb_ref[1:2, :]
        wx_q = xb_ref[2:3, :H]
        b_q = xb_ref[3:4, :H]
        for k in range(C):
            dtk = dch[:, k:k + 1]
            pre_rs = dtk * wx_rs + b_rs
            pre_q = dtk * wx_q + b_q
            dec = [h[m] * edt[m][:, k:k + 1] for m in range(M)]
            for m in range(M):
                aft_ref[m, k, :, :] = h[m]
                bef_ref[m, k, :, :] = dec[m]
            h_comb = dec[0] + dec[1] + dec[2]
            ln_rs = jax.lax.dot(h_comb, wh_rs,
                                preferred_element_type=jnp.float32) + pre_rs
            ln_r = ln_rs[:, :H]
            ln_s = ln_rs[:, H:]
            a = [-(ln_r - LNT[m]) ** 2 for m in range(M)]
            amax = jnp.maximum(jnp.maximum(a[0], a[1]), a[2])
            r = [jnp.exp(a[m] - amax) for m in range(M)]
            rsum = r[0] + r[1] + r[2]
            h_ret = (r[0] * dec[0] + r[1] * dec[1] + r[2] * dec[2]) / rsum
            q = jnp.tanh(jax.lax.dot(h_ret, wh_q,
                                     preferred_element_type=jnp.float32)
                         + pre_q)
            c = [-(ln_s - LNT[m]) ** 2 for m in range(M)]
            cmax = jnp.maximum(jnp.maximum(c[0], c[1]), c[2])
            s = [jnp.exp(c[m] - cmax) for m in range(M)]
            ssum = s[0] + s[1] + s[2]
            h = [dec[m] + (s[m] / ssum) * (q - dec[m]) for m in range(M)]
        for m in range(M):
            hhat_ref[m] = h[m]

    @pl.when(i == NCHUNK)
    def _epilogue():
        dt_last = dch_ref[NCHUNK][:, 0:1]
        for m in range(M):
            hm = hhat_ref[m]
            aft_ref[m, 0, :, :] = hm
            bef_ref[m, 0, :, :] = hm * jnp.exp(dt_last * (-1.0 / TAUS[m]))


def kernel(seq_pads, seq_lens, Wr, br, Ws, bs, Wq, bq):
    sp_t = seq_pads.reshape(B, L).T
    lens = seq_lens.astype(jnp.int32)
    delta_t = _delta_sc(sp_t, lens)

    wh_rs = jnp.concatenate([Wr[1:], Ws[1:]], axis=1)
    wh_q = Wq[1:]
    zpad = jnp.zeros((H,), jnp.float32)
    xb = jnp.stack([
        jnp.concatenate([Wr[0], Ws[0]]),
        jnp.concatenate([br, bs]),
        jnp.concatenate([Wq[0], zpad]),
        jnp.concatenate([bq, zpad]),
    ] + [jnp.zeros((2 * H,), jnp.float32)] * 4, axis=0)

    bef_t, aft_t, delta = pl.pallas_call(
        _scan_body,
        grid=(GRID,),
        in_specs=[
            pl.BlockSpec((GRID * C, B), lambda i: (0, 0)),
            pl.BlockSpec((H, 2 * H), lambda i: (0, 0)),
            pl.BlockSpec((H, H), lambda i: (0, 0)),
            pl.BlockSpec((8, 2 * H), lambda i: (0, 0)),
        ],
        out_specs=[
            pl.BlockSpec((M, C, B, H), lambda i: (0, i, 0, 0)),
            pl.BlockSpec((M, C, B, H), lambda i: (0, i, 0, 0)),
            pl.BlockSpec((B, L + 1), lambda i: (0, 0)),
        ],
        out_shape=[
            jax.ShapeDtypeStruct((M, L + 1, B, H), jnp.float32),
            jax.ShapeDtypeStruct((M, L + 1, B, H), jnp.float32),
            jax.ShapeDtypeStruct((B, L + 1), jnp.float32),
        ],
        scratch_shapes=[
            pltpu.VMEM((M, B, H), jnp.float32),
            pltpu.VMEM((GRID, B, C), jnp.float32),
        ],
        compiler_params=pltpu.CompilerParams(
            dimension_semantics=("arbitrary",)),
    )(delta_t, wh_rs, wh_q, xb)

    befores = jnp.transpose(bef_t, (0, 2, 1, 3))
    afters = jnp.transpose(aft_t, (0, 2, 1, 3))
    return befores, afters, delta[:, :, None]


# speedup vs baseline: 1.0027x; 1.0027x over previous
import functools
import math

import jax
import jax.numpy as jnp
from jax import lax
from jax.experimental import pallas as pl
from jax.experimental.pallas import tpu as pltpu
from jax.experimental.pallas import tpu_sc as plsc

B = 16
L = 512
H = 256
M = 3
T_END = 100.0
TAUS = (1.0, 10.0, 100.0)
LNT = tuple(math.log(t) for t in TAUS)
C = 128
NCHUNK = L // C
GRID = NCHUNK + 1

_SC_INFO = plsc.get_sparse_core_info()
NW = _SC_INFO.num_cores * _SC_INFO.num_subcores
ROWS_W = 24
OUT_ROWS = NW * ROWS_W


def _delta_sc_body(sp_hbm, lens_hbm, out_hbm, sp_v, lens_v, buf_v):
    wid = lax.axis_index("s") * _SC_INFO.num_cores + lax.axis_index("c")
    lo = wid * ROWS_W
    pltpu.sync_copy(sp_hbm, sp_v)
    pltpu.sync_copy(lens_hbm, lens_v)
    lens = lens_v[...]
    lanes = lax.iota(jnp.int32, 16)
    t_last = jnp.zeros((16,), jnp.float32)
    for b in range(16):
        lb = lens[b]
        rowb = sp_v[lb - 1]
        oneh = jnp.clip(1 - jnp.abs(lanes - b), 0, 1).astype(jnp.float32)
        t_last = t_last + oneh * rowb
    fill_last = jnp.full((16,), T_END, jnp.float32) - t_last

    def row(idx, carry):
        j = lo + idx
        jc = jnp.minimum(j, L - 1)
        sp_j = sp_v[jc]
        sp_jm1 = sp_v[jnp.maximum(jc - 1, 0)]
        has_prev = jnp.minimum(j, 1)
        d = sp_j - jnp.full((16,), has_prev, jnp.int32).astype(jnp.float32) * sp_jm1
        a = lens - jnp.full((16,), j, jnp.int32)
        m_lt = jnp.clip(a, 0, 1).astype(jnp.float32)
        m_eq = jnp.clip(1 - jnp.abs(a), 0, 1).astype(jnp.float32)
        delta_j = m_lt * d + m_eq * fill_last - (1.0 - m_lt - m_eq)
        buf_v[idx] = delta_j
        return carry

    lax.fori_loop(0, ROWS_W, row, 0)
    pltpu.sync_copy(buf_v, out_hbm.at[pl.ds(lo, ROWS_W)])


_delta_sc = functools.partial(
    pl.kernel,
    out_type=jax.ShapeDtypeStruct((OUT_ROWS, B), jnp.float32),
    mesh=plsc.VectorSubcoreMesh(core_axis_name="c", subcore_axis_name="s"),
    scratch_types=[
        pltpu.VMEM((L, B), jnp.float32),
        pltpu.VMEM((B,), jnp.int32),
        pltpu.VMEM((ROWS_W, B), jnp.float32),
    ],
)(_delta_sc_body)


def _scan_body(delta_t_ref, whrs_ref, whq_ref, xb_ref,
               bef_ref, aft_ref, delta_ref, hhat_ref, dch_ref):
    i = pl.program_id(0)

    @pl.when(i == 0)
    def _init():
        cols = []
        for j in range(GRID):
            t = jnp.transpose(delta_t_ref[j * C:(j + 1) * C, :])
            dch_ref[j] = t
            cols.append(t)
        delta_ref[...] = jnp.concatenate(cols, axis=1)[:, :L + 1]
        hhat_ref[...] = jnp.zeros((M, B, H), jnp.float32)

    @pl.when(i < NCHUNK)
    def _chunk():
        dch = dch_ref[i]
        edt = [jnp.exp(dch * (-1.0 / TAUS[m])) for m in range(M)]
        h = [hhat_ref[m] for m in range(M)]
        wh_rs = whrs_ref[...]
        wh_q = whq_ref[...]
        wx_rs = xb_ref[0:1, :]
        b_rs = xb_ref[1:2, :]
        wx_q = xb_ref[2:3, :H]
        b_q = xb_ref[3:4, :H]
        for k in range(C):
            dtk = dch[:, k:k + 1]
            pre_rs = dtk * wx_rs + b_rs
            pre_q = dtk * wx_q + b_q
            dec = [h[m] * edt[m][:, k:k + 1] for m in range(M)]
            for m in range(M):
                aft_ref[m, k, :, :] = h[m]
                bef_ref[m, k, :, :] = dec[m]
            h_comb = dec[0] + dec[1] + dec[2]
            ln_rs = jax.lax.dot(h_comb, wh_rs,
                                preferred_element_type=jnp.float32) + pre_rs
            ln_r = ln_rs[:, :H]
            ln_s = ln_rs[:, H:]
            a = [-(ln_r - LNT[m]) ** 2 for m in range(M)]
            amax = jnp.maximum(jnp.maximum(a[0], a[1]), a[2])
            r = [jnp.exp(a[m] - amax) for m in range(M)]
            rsum = r[0] + r[1] + r[2]
            h_ret = (r[0] * dec[0] + r[1] * dec[1] + r[2] * dec[2]) / rsum
            q = jnp.tanh(jax.lax.dot(h_ret, wh_q,
                                     preferred_element_type=jnp.float32)
                         + pre_q)
            c = [-(ln_s - LNT[m]) ** 2 for m in range(M)]
            cmax = jnp.maximum(jnp.maximum(c[0], c[1]), c[2])
            s = [jnp.exp(c[m] - cmax) for m in range(M)]
            ssum = s[0] + s[1] + s[2]
            h = [dec[m] + (s[m] / ssum) * (q - dec[m]) for m in range(M)]
        for m in range(M):
            hhat_ref[m] = h[m]

    @pl.when(i == NCHUNK)
    def _epilogue():
        dt_last = dch_ref[NCHUNK][:, 0:1]
        for m in range(M):
            hm = hhat_ref[m]
            aft_ref[m, 0, :, :] = hm
            bef_ref[m, 0, :, :] = hm * jnp.exp(dt_last * (-1.0 / TAUS[m]))


def kernel(seq_pads, seq_lens, Wr, br, Ws, bs, Wq, bq):
    sp_t = seq_pads.reshape(B, L).T
    lens = seq_lens.astype(jnp.int32)
    delta_t = _delta_sc(sp_t, lens)

    wh_rs = jnp.concatenate([Wr[1:], Ws[1:]], axis=1)
    wh_q = Wq[1:]
    zpad = jnp.zeros((H,), jnp.float32)
    xb = jnp.stack([
        jnp.concatenate([Wr[0], Ws[0]]),
        jnp.concatenate([br, bs]),
        jnp.concatenate([Wq[0], zpad]),
        jnp.concatenate([bq, zpad]),
    ] + [jnp.zeros((2 * H,), jnp.float32)] * 4, axis=0)

    bef_t, aft_t, delta = pl.pallas_call(
        _scan_body,
        grid=(GRID,),
        in_specs=[
            pl.BlockSpec((GRID * C, B), lambda i: (0, 0)),
            pl.BlockSpec((H, 2 * H), lambda i: (0, 0)),
            pl.BlockSpec((H, H), lambda i: (0, 0)),
            pl.BlockSpec((8, 2 * H), lambda i: (0, 0)),
        ],
        out_specs=[
            pl.BlockSpec((M, C, B, H), lambda i: (0, i, 0, 0)),
            pl.BlockSpec((M, C, B, H), lambda i: (0, i, 0, 0)),
            pl.BlockSpec((B, L + 1), lambda i: (0, 0)),
        ],
        out_shape=[
            jax.ShapeDtypeStruct((M, L + 1, B, H), jnp.float32),
            jax.ShapeDtypeStruct((M, L + 1, B, H), jnp.float32),
            jax.ShapeDtypeStruct((B, L + 1), jnp.float32),
        ],
        scratch_shapes=[
            pltpu.VMEM((M, B, H), jnp.float32),
            pltpu.VMEM((GRID, B, C), jnp.float32),
        ],
        compiler_params=pltpu.CompilerParams(
            dimension_semantics=("arbitrary",)),
    )(delta_t, wh_rs, wh_q, xb)

    befores = jnp.transpose(bef_t, (0, 2, 1, 3))
    afters = jnp.transpose(aft_t, (0, 2, 1, 3))
    return befores, afters, delta[:, :, None]


# speedup vs baseline: 1.0027x; 1.0000x over previous
import functools
import math

import jax
import jax.numpy as jnp
from jax import lax
from jax.experimental import pallas as pl
from jax.experimental.pallas import tpu as pltpu
from jax.experimental.pallas import tpu_sc as plsc

B = 16
L = 512
H = 256
M = 3
T_END = 100.0
TAUS = (1.0, 10.0, 100.0)
LNT = tuple(math.log(t) for t in TAUS)
C = 64
NCHUNK = L // C
GRID = NCHUNK + 1

_SC_INFO = plsc.get_sparse_core_info()
NW = _SC_INFO.num_cores * _SC_INFO.num_subcores
ROWS_W = 24
OUT_ROWS = NW * ROWS_W


def _delta_sc_body(sp_hbm, lens_hbm, out_hbm, sp_v, lens_v, buf_v):
    wid = lax.axis_index("s") * _SC_INFO.num_cores + lax.axis_index("c")
    lo = wid * ROWS_W
    pltpu.sync_copy(sp_hbm, sp_v)
    pltpu.sync_copy(lens_hbm, lens_v)
    lens = lens_v[...]
    lanes = lax.iota(jnp.int32, 16)
    t_last = jnp.zeros((16,), jnp.float32)
    for b in range(16):
        lb = lens[b]
        rowb = sp_v[lb - 1]
        oneh = jnp.clip(1 - jnp.abs(lanes - b), 0, 1).astype(jnp.float32)
        t_last = t_last + oneh * rowb
    fill_last = jnp.full((16,), T_END, jnp.float32) - t_last

    def row(idx, carry):
        j = lo + idx
        jc = jnp.minimum(j, L - 1)
        sp_j = sp_v[jc]
        sp_jm1 = sp_v[jnp.maximum(jc - 1, 0)]
        has_prev = jnp.minimum(j, 1)
        d = sp_j - jnp.full((16,), has_prev, jnp.int32).astype(jnp.float32) * sp_jm1
        a = lens - jnp.full((16,), j, jnp.int32)
        m_lt = jnp.clip(a, 0, 1).astype(jnp.float32)
        m_eq = jnp.clip(1 - jnp.abs(a), 0, 1).astype(jnp.float32)
        delta_j = m_lt * d + m_eq * fill_last - (1.0 - m_lt - m_eq)
        buf_v[idx] = delta_j
        return carry

    lax.fori_loop(0, ROWS_W, row, 0)
    pltpu.sync_copy(buf_v, out_hbm.at[pl.ds(lo, ROWS_W)])


_delta_sc = functools.partial(
    pl.kernel,
    out_type=jax.ShapeDtypeStruct((OUT_ROWS, B), jnp.float32),
    mesh=plsc.VectorSubcoreMesh(core_axis_name="c", subcore_axis_name="s"),
    scratch_types=[
        pltpu.VMEM((L, B), jnp.float32),
        pltpu.VMEM((B,), jnp.int32),
        pltpu.VMEM((ROWS_W, B), jnp.float32),
    ],
)(_delta_sc_body)


def _scan_body(delta_t_ref, whrs_ref, whq_ref, xb_ref,
               bef_ref, aft_ref, delta_ref, hhat_ref, dch_ref):
    i = pl.program_id(0)

    @pl.when(i == 0)
    def _init():
        cols = []
        for j in range(GRID):
            t = jnp.transpose(delta_t_ref[j * C:(j + 1) * C, :])
            dch_ref[j] = t
            cols.append(t)
        delta_ref[...] = jnp.concatenate(cols, axis=1)[:, :L + 1]
        hhat_ref[...] = jnp.zeros((M, B, H), jnp.float32)

    @pl.when(i < NCHUNK)
    def _chunk():
        dch = dch_ref[i]
        edt = [jnp.exp(dch * (-1.0 / TAUS[m])) for m in range(M)]
        h = [hhat_ref[m] for m in range(M)]
        wh_rs = whrs_ref[...]
        wh_q = whq_ref[...]
        wx_rs = xb_ref[0:1, :]
        b_rs = xb_ref[1:2, :]
        wx_q = xb_ref[2:3, :H]
        b_q = xb_ref[3:4, :H]
        for k in range(C):
            dtk = dch[:, k:k + 1]
            pre_rs = dtk * wx_rs + b_rs
            pre_q = dtk * wx_q + b_q
            dec = [h[m] * edt[m][:, k:k + 1] for m in range(M)]
            for m in range(M):
                aft_ref[m, k, :, :] = h[m]
                bef_ref[m, k, :, :] = dec[m]
            h_comb = dec[0] + dec[1] + dec[2]
            ln_rs = jax.lax.dot(h_comb, wh_rs,
                                preferred_element_type=jnp.float32) + pre_rs
            ln_r = ln_rs[:, :H]
            ln_s = ln_rs[:, H:]
            a = [-(ln_r - LNT[m]) ** 2 for m in range(M)]
            amax = jnp.maximum(jnp.maximum(a[0], a[1]), a[2])
            r = [jnp.exp(a[m] - amax) for m in range(M)]
            rsum = r[0] + r[1] + r[2]
            h_ret = (r[0] * dec[0] + r[1] * dec[1] + r[2] * dec[2]) / rsum
            q = jnp.tanh(jax.lax.dot(h_ret, wh_q,
                                     preferred_element_type=jnp.float32)
                         + pre_q)
            c = [-(ln_s - LNT[m]) ** 2 for m in range(M)]
            cmax = jnp.maximum(jnp.maximum(c[0], c[1]), c[2])
            s = [jnp.exp(c[m] - cmax) for m in range(M)]
            ssum = s[0] + s[1] + s[2]
            h = [dec[m] + (s[m] / ssum) * (q - dec[m]) for m in range(M)]
        for m in range(M):
            hhat_ref[m] = h[m]

    @pl.when(i == NCHUNK)
    def _epilogue():
        dt_last = dch_ref[NCHUNK][:, 0:1]
        for m in range(M):
            hm = hhat_ref[m]
            aft_ref[m, 0, :, :] = hm
            bef_ref[m, 0, :, :] = hm * jnp.exp(dt_last * (-1.0 / TAUS[m]))


def kernel(seq_pads, seq_lens, Wr, br, Ws, bs, Wq, bq):
    sp_t = seq_pads.reshape(B, L).T
    lens = seq_lens.astype(jnp.int32)
    delta_t = _delta_sc(sp_t, lens)

    wh_rs = jnp.concatenate([Wr[1:], Ws[1:]], axis=1)
    wh_q = Wq[1:]
    zpad = jnp.zeros((H,), jnp.float32)
    xb = jnp.stack([
        jnp.concatenate([Wr[0], Ws[0]]),
        jnp.concatenate([br, bs]),
        jnp.concatenate([Wq[0], zpad]),
        jnp.concatenate([bq, zpad]),
    ] + [jnp.zeros((2 * H,), jnp.float32)] * 4, axis=0)

    bef_t, aft_t, delta = pl.pallas_call(
        _scan_body,
        grid=(GRID,),
        in_specs=[
            pl.BlockSpec((GRID * C, B), lambda i: (0, 0)),
            pl.BlockSpec((H, 2 * H), lambda i: (0, 0)),
            pl.BlockSpec((H, H), lambda i: (0, 0)),
            pl.BlockSpec((8, 2 * H), lambda i: (0, 0)),
        ],
        out_specs=[
            pl.BlockSpec((M, C, B, H), lambda i: (0, i, 0, 0)),
            pl.BlockSpec((M, C, B, H), lambda i: (0, i, 0, 0)),
            pl.BlockSpec((B, L + 1), lambda i: (0, 0)),
        ],
        out_shape=[
            jax.ShapeDtypeStruct((M, L + 1, B, H), jnp.float32),
            jax.ShapeDtypeStruct((M, L + 1, B, H), jnp.float32),
            jax.ShapeDtypeStruct((B, L + 1), jnp.float32),
        ],
        scratch_shapes=[
            pltpu.VMEM((M, B, H), jnp.float32),
            pltpu.VMEM((GRID, B, C), jnp.float32),
        ],
        compiler_params=pltpu.CompilerParams(
            dimension_semantics=("arbitrary",)),
    )(delta_t, wh_rs, wh_q, xb)

    befores = jnp.transpose(bef_t, (0, 2, 1, 3))
    afters = jnp.transpose(aft_t, (0, 2, 1, 3))
    return befores, afters, delta[:, :, None]


# speedup vs baseline: 1.0252x; 1.0224x over previous
import functools
import math

import jax
import jax.numpy as jnp
from jax import lax
from jax.experimental import pallas as pl
from jax.experimental.pallas import tpu as pltpu
from jax.experimental.pallas import tpu_sc as plsc

B = 16
L = 512
H = 256
M = 3
T_END = 100.0
TAUS = (1.0, 10.0, 100.0)
LNT = tuple(math.log(t) for t in TAUS)
C = 64
NCHUNK = L // C
GRID = NCHUNK + 1

_SC_INFO = plsc.get_sparse_core_info()
NW = _SC_INFO.num_cores * _SC_INFO.num_subcores
ROWS_W = 24
OUT_ROWS = NW * ROWS_W


def _delta_sc_body(sp_hbm, lens_hbm, out_hbm, sp_v, lens_v, buf_v):
    wid = lax.axis_index("s") * _SC_INFO.num_cores + lax.axis_index("c")
    lo = wid * ROWS_W
    pltpu.sync_copy(sp_hbm, sp_v)
    pltpu.sync_copy(lens_hbm, lens_v)
    lens = lens_v[...]
    lanes = lax.iota(jnp.int32, 16)
    t_last = jnp.zeros((16,), jnp.float32)
    for b in range(16):
        lb = lens[b]
        rowb = sp_v[lb - 1]
        oneh = jnp.clip(1 - jnp.abs(lanes - b), 0, 1).astype(jnp.float32)
        t_last = t_last + oneh * rowb
    fill_last = jnp.full((16,), T_END, jnp.float32) - t_last

    def row(idx, carry):
        j = lo + idx
        jc = jnp.minimum(j, L - 1)
        sp_j = sp_v[jc]
        sp_jm1 = sp_v[jnp.maximum(jc - 1, 0)]
        has_prev = jnp.minimum(j, 1)
        d = sp_j - jnp.full((16,), has_prev, jnp.int32).astype(jnp.float32) * sp_jm1
        a = lens - jnp.full((16,), j, jnp.int32)
        m_lt = jnp.clip(a, 0, 1).astype(jnp.float32)
        m_eq = jnp.clip(1 - jnp.abs(a), 0, 1).astype(jnp.float32)
        delta_j = m_lt * d + m_eq * fill_last - (1.0 - m_lt - m_eq)
        buf_v[idx] = delta_j
        return carry

    lax.fori_loop(0, ROWS_W, row, 0)
    pltpu.sync_copy(buf_v, out_hbm.at[pl.ds(lo, ROWS_W)])


_delta_sc = functools.partial(
    pl.kernel,
    out_type=jax.ShapeDtypeStruct((OUT_ROWS, B), jnp.float32),
    mesh=plsc.VectorSubcoreMesh(core_axis_name="c", subcore_axis_name="s"),
    scratch_types=[
        pltpu.VMEM((L, B), jnp.float32),
        pltpu.VMEM((B,), jnp.int32),
        pltpu.VMEM((ROWS_W, B), jnp.float32),
    ],
)(_delta_sc_body)


def _scan_body(delta_t_ref, whrs_ref, whq_ref, xb_ref,
               bef_ref, aft_ref, delta_ref, hhat_ref, dch_ref):
    i = pl.program_id(0)

    @pl.when(i == 0)
    def _init():
        cols = []
        for j in range(GRID):
            t = jnp.transpose(delta_t_ref[j * C:(j + 1) * C, :])
            dch_ref[j] = t
            cols.append(t)
        delta_ref[...] = jnp.concatenate(cols, axis=1)[:, :L + 1]
        hhat_ref[...] = jnp.zeros((M, B, H), jnp.float32)

    @pl.when(i < NCHUNK)
    def _chunk():
        dch = dch_ref[i]
        edt = [jnp.exp(dch * (-1.0 / TAUS[m])) for m in range(M)]
        h = [hhat_ref[m] for m in range(M)]
        wh_r = whrs_ref[:, :H]
        wh_s = whrs_ref[:, H:]
        wh_q = whq_ref[...]
        wx_rs = xb_ref[0:1, :]
        b_rs = xb_ref[1:2, :]
        wx_q = xb_ref[2:3, :H]
        b_q = xb_ref[3:4, :H]
        for k in range(C):
            dtk = dch[:, k:k + 1]
            pre_rs = dtk * wx_rs + b_rs
            pre_q = dtk * wx_q + b_q
            dec = [h[m] * edt[m][:, k:k + 1] for m in range(M)]
            for m in range(M):
                aft_ref[m, k, :, :] = h[m]
                bef_ref[m, k, :, :] = dec[m]
            h_comb = dec[0] + dec[1] + dec[2]
            ln_r = jax.lax.dot(h_comb, wh_r,
                               preferred_element_type=jnp.float32) \
                + pre_rs[:, :H]
            a = [-(ln_r - LNT[m]) ** 2 for m in range(M)]
            amax = jnp.maximum(jnp.maximum(a[0], a[1]), a[2])
            r = [jnp.exp(a[m] - amax) for m in range(M)]
            rsum = r[0] + r[1] + r[2]
            h_ret = (r[0] * dec[0] + r[1] * dec[1] + r[2] * dec[2]) / rsum
            q = jnp.tanh(jax.lax.dot(h_ret, wh_q,
                                     preferred_element_type=jnp.float32)
                         + pre_q)
            ln_s = jax.lax.dot(h_comb, wh_s,
                               preferred_element_type=jnp.float32) \
                + pre_rs[:, H:]
            c = [-(ln_s - LNT[m]) ** 2 for m in range(M)]
            cmax = jnp.maximum(jnp.maximum(c[0], c[1]), c[2])
            s = [jnp.exp(c[m] - cmax) for m in range(M)]
            ssum = s[0] + s[1] + s[2]
            h = [dec[m] + (s[m] / ssum) * (q - dec[m]) for m in range(M)]
        for m in range(M):
            hhat_ref[m] = h[m]

    @pl.when(i == NCHUNK)
    def _epilogue():
        dt_last = dch_ref[NCHUNK][:, 0:1]
        for m in range(M):
            hm = hhat_ref[m]
            aft_ref[m, 0, :, :] = hm
            bef_ref[m, 0, :, :] = hm * jnp.exp(dt_last * (-1.0 / TAUS[m]))


def kernel(seq_pads, seq_lens, Wr, br, Ws, bs, Wq, bq):
    sp_t = seq_pads.reshape(B, L).T
    lens = seq_lens.astype(jnp.int32)
    delta_t = _delta_sc(sp_t, lens)

    wh_rs = jnp.concatenate([Wr[1:], Ws[1:]], axis=1)
    wh_q = Wq[1:]
    zpad = jnp.zeros((H,), jnp.float32)
    xb = jnp.stack([
        jnp.concatenate([Wr[0], Ws[0]]),
        jnp.concatenate([br, bs]),
        jnp.concatenate([Wq[0], zpad]),
        jnp.concatenate([bq, zpad]),
    ] + [jnp.zeros((2 * H,), jnp.float32)] * 4, axis=0)

    bef_t, aft_t, delta = pl.pallas_call(
        _scan_body,
        grid=(GRID,),
        in_specs=[
            pl.BlockSpec((GRID * C, B), lambda i: (0, 0)),
            pl.BlockSpec((H, 2 * H), lambda i: (0, 0)),
            pl.BlockSpec((H, H), lambda i: (0, 0)),
            pl.BlockSpec((8, 2 * H), lambda i: (0, 0)),
        ],
        out_specs=[
            pl.BlockSpec((M, C, B, H), lambda i: (0, i, 0, 0)),
            pl.BlockSpec((M, C, B, H), lambda i: (0, i, 0, 0)),
            pl.BlockSpec((B, L + 1), lambda i: (0, 0)),
        ],
        out_shape=[
            jax.ShapeDtypeStruct((M, L + 1, B, H), jnp.float32),
            jax.ShapeDtypeStruct((M, L + 1, B, H), jnp.float32),
            jax.ShapeDtypeStruct((B, L + 1), jnp.float32),
        ],
        scratch_shapes=[
            pltpu.VMEM((M, B, H), jnp.float32),
            pltpu.VMEM((GRID, B, C), jnp.float32),
        ],
        compiler_params=pltpu.CompilerParams(
            dimension_semantics=("arbitrary",)),
    )(delta_t, wh_rs, wh_q, xb)

    befores = jnp.transpose(bef_t, (0, 2, 1, 3))
    afters = jnp.transpose(aft_t, (0, 2, 1, 3))
    return befores, afters, delta[:, :, None]
